# Initial kernel scaffold; baseline (speedup 1.0000x reference)
import jax
import jax.numpy as jnp
from jax.experimental import pallas as pl


def kernel(extended_coord, extended_atype, nlist, mapping, type_table, ri_w0, ri_b0, ri_w1, ri_b1, ri_w2, ri_b2, g1w, g2i_w, wg2, wattn, wg1, bg1):
    raise NotImplementedError("write your pallas kernel here")


# speedup vs baseline: 2.9598x; 2.9598x over previous
import functools

import jax
import jax.numpy as jnp
from jax import lax
from jax.experimental import pallas as pl
from jax.experimental.pallas import tpu as pltpu
from jax.experimental.pallas import tpu_sc as plsc

NLOC, NALL = 10000, 12000
NNEI, NNEI2 = 120, 40
NTYPES, TEBD = 8, 8
N0, N1, N2 = 25, 50, 100
AXIS = 12
G1, G2D, NL = 128, 32, 6
RC1, RS1 = 9.0, 8.0
RC2, RS2 = 4.0, 3.5

A = 64
NLOCP = 10240
NBLK = NLOCP // A
NC, NS = 2, 16
NW = NC * NS
E1 = NLOCP * NNEI
W1 = E1 // NW
CH1, K1 = 1920, 20
E2 = NLOCP * NNEI2
W2 = E2 // NW
CH2, K2 = 1600, 8


def _swfn(r, rs, rc):
    u = jnp.clip((r - rs) / (rc - rs), 0.0, 1.0)
    return u * u * u * (-6.0 * u * u + 15.0 * u - 10.0) + 1.0


def _table_body(coord_ref, atype_ref, tt_ref, out_ref):
    at = atype_ref[...]
    oh = (at == lax.broadcasted_iota(jnp.int32, (NALL, NTYPES), 1)).astype(jnp.float32)
    tebd = jnp.dot(oh, tt_ref[...], preferred_element_type=jnp.float32)
    out_ref[...] = jnp.concatenate(
        [coord_ref[...], tebd, jnp.zeros((NALL, 5), jnp.float32)], axis=1)


def _build_table16(coord, atype2d, type_table):
    return pl.pallas_call(
        _table_body,
        out_shape=jax.ShapeDtypeStruct((NALL, 16), jnp.float32),
    )(coord, atype2d, type_table)


def _sc_gather_rows(T, D, E, CH, K, dtype):
    mesh = plsc.VectorSubcoreMesh(core_axis_name="c", subcore_axis_name="s",
                                  num_cores=NC, num_subcores=NS)

    def body(tab_hbm, idx_hbm, out_hbm, idx_v, rows_v, sem):
        wid = lax.axis_index("s") * NC + lax.axis_index("c")
        for k in range(K):
            base = wid * (CH * K) + k * CH
            pltpu.sync_copy(idx_hbm.at[pl.ds(base, CH)], idx_v)
            pltpu.async_copy(tab_hbm.at[idx_v], rows_v, sem).wait()
            pltpu.sync_copy(rows_v, out_hbm.at[pl.ds(base, CH)])

    return pl.kernel(
        body,
        out_type=jax.ShapeDtypeStruct((E, D), dtype),
        mesh=mesh,
        compiler_params=pltpu.CompilerParams(use_tc_tiling_on_sc=False),
        scratch_types=[
            pltpu.VMEM((CH,), jnp.int32),
            pltpu.VMEM((CH, D), dtype),
            pltpu.SemaphoreType.DMA,
        ],
    )


def _sc_gather_scalar(T, E, CH, K, dtype):
    mesh = plsc.VectorSubcoreMesh(core_axis_name="c", subcore_axis_name="s",
                                  num_cores=NC, num_subcores=NS)

    def body(tab_hbm, idx_hbm, out_hbm, idx_v, val_v, sem):
        wid = lax.axis_index("s") * NC + lax.axis_index("c")
        for k in range(K):
            base = wid * (CH * K) + k * CH
            pltpu.sync_copy(idx_hbm.at[pl.ds(base, CH)], idx_v)
            pltpu.async_copy(tab_hbm.at[idx_v], val_v, sem).wait()
            pltpu.sync_copy(val_v, out_hbm.at[pl.ds(base, CH)])

    return pl.kernel(
        body,
        out_type=jax.ShapeDtypeStruct((E,), dtype),
        mesh=mesh,
        compiler_params=pltpu.CompilerParams(use_tc_tiling_on_sc=False),
        scratch_types=[
            pltpu.VMEM((CH,), jnp.int32),
            pltpu.VMEM((CH,), dtype),
            pltpu.SemaphoreType.DMA,
        ],
    )


def _stage1_body(rows_ref, own_ref, w0_ref, b0_ref, w1_ref, b1_ref, w2_ref,
                 b2_ref, g1w3_ref, g2iw_ref, wg20_ref,
                 g1_ref, p_ref, g2_ref, sw2_ref, env2_ref):
    rows = rows_ref[...]
    own = own_ref[...]
    coord_j = rows[:, :, 0:3]
    tebd_j = rows[:, :, 3:3 + TEBD]
    coord_i = own[:, 0:3]
    tebd_i = own[:, 3:3 + TEBD]
    rij = coord_j - coord_i[:, None, :]
    r = jnp.sqrt(jnp.sum(rij * rij, axis=-1) + 1e-6)
    sw = _swfn(r, RS1, RC1)
    s = sw / r
    srij = s[..., None] * rij / r[..., None]
    emb_in = jnp.concatenate(
        [s[..., None],
         jnp.broadcast_to(tebd_i[:, None, :], (A, NNEI, TEBD)),
         tebd_j], axis=-1)
    h = jnp.tanh(jnp.dot(emb_in.reshape(A * NNEI, 1 + 2 * TEBD), w0_ref[...],
                         preferred_element_type=jnp.float32) + b0_ref[...])
    h = jnp.tanh(jnp.dot(h, w1_ref[...], preferred_element_type=jnp.float32)
                 + b1_ref[...])
    h = jnp.tanh(jnp.dot(h, w2_ref[...], preferred_element_type=jnp.float32)
                 + b2_ref[...])
    gg = h.reshape(A, NNEI, N2) * sw[..., None]
    env = jnp.concatenate([s[..., None], srij], axis=-1)
    grr = lax.dot_general(env, gg, (((1,), (1,)), ((0,), (0,))),
                          preferred_element_type=jnp.float32) * (1.0 / NNEI)
    grr_ax = grr[:, :, :AXIS]
    desc2 = lax.dot_general(grr_ax, grr, (((1,), (1,)), ((0,), (0,))),
                            preferred_element_type=jnp.float32)
    g1 = jnp.zeros((A, G1), jnp.float32)
    for x in range(AXIS):
        g1 = g1 + jnp.dot(desc2[:, x, :], g1w3_ref[x],
                          preferred_element_type=jnp.float32)
    g1_ref[...] = g1
    p_ref[...] = jnp.dot(g1, wg20_ref[...], preferred_element_type=jnp.float32)
    rij2 = rij[:, :NNEI2, :]
    r2 = r[:, :NNEI2]
    sw2 = _swfn(r2, RS2, RC2)
    s2 = sw2 / r2
    env2 = jnp.concatenate(
        [s2[..., None], s2[..., None] * rij2 / r2[..., None]], axis=-1)
    g2_ref[...] = jnp.tanh(
        jnp.dot(env2.reshape(A * NNEI2, 4), g2iw_ref[...],
                preferred_element_type=jnp.float32)).reshape(A, NNEI2, G2D)
    sw2_ref[...] = sw2
    env2_ref[...] = env2


def _stage1_call(rows16, own16, w0, b0, w1, b1, w2, b2, g1w3, g2iw, wg20):
    full = lambda a: pl.BlockSpec(a.shape, lambda i: (0,) * a.ndim)
    return pl.pallas_call(
        _stage1_body,
        grid=(NBLK,),
        in_specs=[
            pl.BlockSpec((A, NNEI, 16), lambda i: (i, 0, 0)),
            pl.BlockSpec((A, 16), lambda i: (i, 0)),
            full(w0), full(b0), full(w1), full(b1), full(w2), full(b2),
            full(g1w3), full(g2iw), full(wg20),
        ],
        out_specs=[
            pl.BlockSpec((A, G1), lambda i: (i, 0)),
            pl.BlockSpec((A, G2D), lambda i: (i, 0)),
            pl.BlockSpec((A, NNEI2, G2D), lambda i: (i, 0, 0)),
            pl.BlockSpec((A, NNEI2), lambda i: (i, 0)),
            pl.BlockSpec((A, NNEI2, 4), lambda i: (i, 0, 0)),
        ],
        out_shape=[
            jax.ShapeDtypeStruct((NLOCP, G1), jnp.float32),
            jax.ShapeDtypeStruct((NLOCP, G2D), jnp.float32),
            jax.ShapeDtypeStruct((NLOCP, NNEI2, G2D), jnp.float32),
            jax.ShapeDtypeStruct((NLOCP, NNEI2), jnp.float32),
            jax.ShapeDtypeStruct((NLOCP, NNEI2, 4), jnp.float32),
        ],
    )(rows16, own16, w0, b0, w1, b1, w2, b2, g1w3, g2iw, wg20)


def _layer_body(last, g1_ref, p_ref, pj_ref, g2_ref, sw2_ref, env2_ref,
                wattn_ref, wg1a_ref, wg1b_ref, wg1c_ref, bg1_ref, wg2n_ref,
                *out_refs):
    g1 = g1_ref[...]
    p = p_ref[...]
    pj = pj_ref[...]
    g2 = g2_ref[...]
    sw2 = sw2_ref[...]
    env2 = env2_ref[...]
    g2a = g2 + jnp.tanh(p[:, None, :] + pj) * sw2[..., None]
    q = jnp.dot(g2a.reshape(A * NNEI2, G2D), wattn_ref[...],
                preferred_element_type=jnp.float32).reshape(A, NNEI2, G2D)
    scores = lax.dot_general(q, g2a, (((2,), (2,)), ((0,), (0,))),
                             preferred_element_type=jnp.float32) * (
                                 1.0 / (G2D ** 0.5))
    mx = jnp.max(scores, axis=-1, keepdims=True)
    e = jnp.exp(scores - mx)
    att = e / jnp.sum(e, axis=-1, keepdims=True)
    g2b = g2a + lax.dot_general(att, g2a, (((2,), (1,)), ((0,), (0,))),
                                preferred_element_type=jnp.float32)
    g2m = jnp.mean(g2b * sw2[..., None], axis=1)
    acc = (jnp.dot(g1, wg1a_ref[...], preferred_element_type=jnp.float32)
           + jnp.dot(g2m, wg1b_ref[...], preferred_element_type=jnp.float32)
           + bg1_ref[...])
    for i in range(4):
        grrg_i = jnp.sum(env2[:, :, i:i + 1] * g2b, axis=1) * (1.0 / NNEI2)
        acc = acc + jnp.dot(grrg_i, wg1c_ref[i],
                            preferred_element_type=jnp.float32)
    g1n = g1 + jnp.tanh(acc)
    out_refs[0][...] = g1n
    if not last:
        out_refs[1][...] = jnp.dot(g1n, wg2n_ref[...],
                                   preferred_element_type=jnp.float32)
        out_refs[2][...] = g2b


def _layer_call(last, g1, p, pj, g2, sw2, env2, wattn, wg1a, wg1b, wg1c,
                bg1r, wg2n):
    full = lambda a: pl.BlockSpec(a.shape, lambda i: (0,) * a.ndim)
    out_specs = [pl.BlockSpec((A, G1), lambda i: (i, 0))]
    out_shape = [jax.ShapeDtypeStruct((NLOCP, G1), jnp.float32)]
    if not last:
        out_specs += [
            pl.BlockSpec((A, G2D), lambda i: (i, 0)),
            pl.BlockSpec((A, NNEI2, G2D), lambda i: (i, 0, 0)),
        ]
        out_shape += [
            jax.ShapeDtypeStruct((NLOCP, G2D), jnp.float32),
            jax.ShapeDtypeStruct((NLOCP, NNEI2, G2D), jnp.float32),
        ]
    return pl.pallas_call(
        functools.partial(_layer_body, last),
        grid=(NBLK,),
        in_specs=[
            pl.BlockSpec((A, G1), lambda i: (i, 0)),
            pl.BlockSpec((A, G2D), lambda i: (i, 0)),
            pl.BlockSpec((A, NNEI2, G2D), lambda i: (i, 0, 0)),
            pl.BlockSpec((A, NNEI2, G2D), lambda i: (i, 0, 0)),
            pl.BlockSpec((A, NNEI2), lambda i: (i, 0)),
            pl.BlockSpec((A, NNEI2, 4), lambda i: (i, 0, 0)),
            full(wattn), full(wg1a), full(wg1b), full(wg1c), full(bg1r),
            full(wg2n),
        ],
        out_specs=out_specs,
        out_shape=out_shape,
    )(g1, p, pj, g2, sw2, env2, wattn, wg1a, wg1b, wg1c, bg1r, wg2n)


def kernel(extended_coord, extended_atype, nlist, mapping, type_table, ri_w0,
           ri_b0, ri_w1, ri_b1, ri_w2, ri_b2, g1w, g2i_w, wg2, wattn, wg1,
           bg1):
    coord = extended_coord[0]
    atype2d = extended_atype[0].astype(jnp.int32).reshape(NALL, 1)
    nl = nlist[0].astype(jnp.int32)
    nlp = jnp.pad(nl, ((0, NLOCP - NLOC), (0, 0)))
    idx1 = nlp.reshape(-1)
    idx2 = nlp[:, :NNEI2].reshape(-1)
    mp = mapping[0].astype(jnp.int32)

    table16 = _build_table16(coord, atype2d, type_table)
    rows16 = _sc_gather_rows(NALL, 16, E1, CH1, K1, jnp.float32)(table16, idx1)
    cidx = _sc_gather_scalar(NALL, E2, CH2, K2, jnp.int32)(mp, idx2)

    own16 = jnp.pad(table16[:NLOC], ((0, NLOCP - NLOC), (0, 0)))
    g1w3 = g1w.reshape(N2, AXIS, G1).transpose(1, 0, 2)
    b0r, b1r, b2r = (b.reshape(1, -1) for b in (ri_b0, ri_b1, ri_b2))

    g1, p, g2, sw2, env2 = _stage1_call(
        rows16.reshape(NLOCP, NNEI, 16), own16,
        ri_w0, b0r, ri_w1, b1r, ri_w2, b2r, g1w3, g2i_w, wg2[0])

    pgather = _sc_gather_rows(NLOCP, G2D, E2, CH2, K2, jnp.float32)
    for ll in range(NL):
        pj = pgather(p, cidx).reshape(NLOCP, NNEI2, G2D)
        last = ll == NL - 1
        wg1a = wg1[ll][:G1]
        wg1b = wg1[ll][G1:G1 + G2D]
        wg1c = wg1[ll][G1 + G2D:].reshape(4, G2D, G1)
        wg2n = wg2[ll + 1] if not last else wg2[0]
        outs = _layer_call(last, g1, p, pj, g2, sw2, env2, wattn[ll], wg1a,
                           wg1b, wg1c, bg1[ll].reshape(1, G1), wg2n)
        if last:
            (g1,) = outs
        else:
            g1, p, g2 = outs

    out = jnp.concatenate([g1[:NLOC], table16[:NLOC, 3:3 + TEBD]], axis=-1)
    return out[None]


# speedup vs baseline: 3.4185x; 1.1550x over previous
import functools

import jax
import jax.numpy as jnp
from jax import lax
from jax.experimental import pallas as pl
from jax.experimental.pallas import tpu as pltpu
from jax.experimental.pallas import tpu_sc as plsc

NLOC, NALL = 10000, 12000
NNEI, NNEI2 = 120, 40
NTYPES, TEBD = 8, 8
N0, N1, N2 = 25, 50, 100
AXIS = 12
G1, G2D, NL = 128, 32, 6
RC1, RS1 = 9.0, 8.0
RC2, RS2 = 4.0, 3.5

AS = 64
AL = 128
NLOCP = 10240
NC, NS = 2, 16
NW = NC * NS
E1 = NLOCP * NNEI
CH1, K1 = 1920, 20
E2 = NLOCP * NNEI2
CH2, K2 = 1600, 8


def _swfn(r, rs, rc):
    u = jnp.clip((r - rs) / (rc - rs), 0.0, 1.0)
    return u * u * u * (-6.0 * u * u + 15.0 * u - 10.0) + 1.0


def _table_body(coord_ref, atype_ref, tt_ref, out_ref):
    at = atype_ref[...]
    oh = (at == lax.broadcasted_iota(jnp.int32, (NALL, NTYPES), 1)).astype(jnp.float32)
    tebd = jnp.dot(oh, tt_ref[...], preferred_element_type=jnp.float32)
    out_ref[...] = jnp.concatenate(
        [coord_ref[...], tebd, jnp.zeros((NALL, 5), jnp.float32)], axis=1)


def _build_table16(coord, atype2d, type_table):
    return pl.pallas_call(
        _table_body,
        out_shape=jax.ShapeDtypeStruct((NALL, 16), jnp.float32),
    )(coord, atype2d, type_table)


def _sc_mesh():
    return plsc.VectorSubcoreMesh(core_axis_name="c", subcore_axis_name="s",
                                  num_cores=NC, num_subcores=NS)


def _sc_gather_prep():
    def body(tab_hbm, idx1_hbm, map_hbm, idx2_hbm, rows_hbm, cidx_hbm,
             idx_v, rows_v, idx2_v, cidx_v, sem):
        wid = lax.axis_index("s") * NC + lax.axis_index("c")
        for k in range(K1):
            base = wid * (CH1 * K1) + k * CH1
            pltpu.sync_copy(idx1_hbm.at[pl.ds(base, CH1)], idx_v)
            pltpu.async_copy(tab_hbm.at[idx_v], rows_v, sem).wait()
            pltpu.sync_copy(rows_v, rows_hbm.at[pl.ds(base, CH1)])
        for k in range(K2):
            base = wid * (CH2 * K2) + k * CH2
            pltpu.sync_copy(idx2_hbm.at[pl.ds(base, CH2)], idx2_v)
            pltpu.async_copy(map_hbm.at[idx2_v], cidx_v, sem).wait()
            pltpu.sync_copy(cidx_v, cidx_hbm.at[pl.ds(base, CH2)])

    return pl.kernel(
        body,
        out_type=[jax.ShapeDtypeStruct((E1, 16), jnp.float32),
                  jax.ShapeDtypeStruct((E2,), jnp.int32)],
        mesh=_sc_mesh(),
        compiler_params=pltpu.CompilerParams(use_tc_tiling_on_sc=False),
        scratch_types=[
            pltpu.VMEM((CH1,), jnp.int32),
            pltpu.VMEM((CH1, 16), jnp.float32),
            pltpu.VMEM((CH2,), jnp.int32),
            pltpu.VMEM((CH2,), jnp.int32),
            pltpu.SemaphoreType.DMA,
        ],
    )


def _sc_gather_p():
    def body(tab_hbm, idx_hbm, out_hbm, idx_v, rows_v0, rows_v1, gsem, ssem):
        wid = lax.axis_index("s") * NC + lax.axis_index("c")
        rows = (rows_v0, rows_v1)
        base0 = wid * (CH2 * K2)
        pltpu.sync_copy(idx_hbm.at[pl.ds(base0, CH2 * K2)], idx_v)
        gh = {0: pltpu.async_copy(tab_hbm.at[idx_v.at[pl.ds(0, CH2)]],
                                  rows[0], gsem)}
        sh = {}
        for k in range(K2):
            b = k % 2
            gh[k].wait()
            if k + 1 < K2:
                if k >= 1:
                    sh[k - 1].wait()
                gh[k + 1] = pltpu.async_copy(
                    tab_hbm.at[idx_v.at[pl.ds((k + 1) * CH2, CH2)]],
                    rows[1 - b], gsem)
            sh[k] = pltpu.async_copy(
                rows[b], out_hbm.at[pl.ds(base0 + k * CH2, CH2)], ssem)
        sh[K2 - 2].wait()
        sh[K2 - 1].wait()

    return pl.kernel(
        body,
        out_type=jax.ShapeDtypeStruct((E2, G2D), jnp.float32),
        mesh=_sc_mesh(),
        compiler_params=pltpu.CompilerParams(use_tc_tiling_on_sc=False),
        scratch_types=[
            pltpu.VMEM((CH2 * K2,), jnp.int32),
            pltpu.VMEM((CH2, G2D), jnp.float32),
            pltpu.VMEM((CH2, G2D), jnp.float32),
            pltpu.SemaphoreType.DMA,
            pltpu.SemaphoreType.DMA,
        ],
    )


def _stage1_body(rows_ref, own_ref, w0_ref, b0_ref, w1_ref, b1_ref, w2_ref,
                 b2_ref, g1w3_ref, g2iw_ref, wg20_ref,
                 g1_ref, p_ref, g2_ref, env5_ref):
    rows = rows_ref[...]
    own = own_ref[...]
    coord_j = rows[:, :, 0:3]
    tebd_j = rows[:, :, 3:3 + TEBD]
    coord_i = own[:, 0:3]
    tebd_i = own[:, 3:3 + TEBD]
    rij = coord_j - coord_i[:, None, :]
    r = jnp.sqrt(jnp.sum(rij * rij, axis=-1, keepdims=True) + 1e-6)
    sw = _swfn(r, RS1, RC1)
    s = sw / r
    srij = s * rij / r
    emb_in = jnp.concatenate(
        [s, jnp.broadcast_to(tebd_i[:, None, :], (AS, NNEI, TEBD)), tebd_j],
        axis=-1)
    h = jnp.tanh(jnp.dot(emb_in.reshape(AS * NNEI, 1 + 2 * TEBD), w0_ref[...],
                         preferred_element_type=jnp.float32) + b0_ref[...])
    h = jnp.tanh(jnp.dot(h, w1_ref[...], preferred_element_type=jnp.float32)
                 + b1_ref[...])
    h = jnp.tanh(jnp.dot(h, w2_ref[...], preferred_element_type=jnp.float32)
                 + b2_ref[...])
    gg = h.reshape(AS, NNEI, N2) * sw
    env = jnp.concatenate([s, srij], axis=-1)
    grr = lax.dot_general(env, gg, (((1,), (1,)), ((0,), (0,))),
                          preferred_element_type=jnp.float32) * (1.0 / NNEI)
    grr_ax = grr[:, :, :AXIS]
    desc2 = lax.dot_general(grr_ax, grr, (((1,), (1,)), ((0,), (0,))),
                            preferred_element_type=jnp.float32)
    g1 = jnp.zeros((AS, G1), jnp.float32)
    for x in range(AXIS):
        g1 = g1 + jnp.dot(desc2[:, x, :], g1w3_ref[x],
                          preferred_element_type=jnp.float32)
    g1_ref[...] = g1
    p_ref[...] = jnp.dot(g1, wg20_ref[...], preferred_element_type=jnp.float32)
    rij2 = rij[:, :NNEI2, :]
    r2 = r[:, :NNEI2, :]
    sw2 = _swfn(r2, RS2, RC2)
    s2 = sw2 / r2
    env2 = jnp.concatenate([s2, s2 * rij2 / r2], axis=-1)
    env5_ref[...] = jnp.concatenate([sw2, env2], axis=-1)
    g2_ref[...] = jnp.tanh(
        jnp.dot(env2.reshape(AS * NNEI2, 4), g2iw_ref[...],
                preferred_element_type=jnp.float32)).reshape(AS, NNEI2, G2D)


def _stage1_call(rows16, own16, w0, b0, w1, b1, w2, b2, g1w3, g2iw, wg20):
    full = lambda a: pl.BlockSpec(a.shape, lambda i: (0,) * a.ndim)
    nblk = NLOCP // AS
    return pl.pallas_call(
        _stage1_body,
        grid=(nblk,),
        in_specs=[
            pl.BlockSpec((AS, NNEI, 16), lambda i: (i, 0, 0)),
            pl.BlockSpec((AS, 16), lambda i: (i, 0)),
            full(w0), full(b0), full(w1), full(b1), full(w2), full(b2),
            full(g1w3), full(g2iw), full(wg20),
        ],
        out_specs=[
            pl.BlockSpec((AS, G1), lambda i: (i, 0)),
            pl.BlockSpec((AS, G2D), lambda i: (i, 0)),
            pl.BlockSpec((AS, NNEI2, G2D), lambda i: (i, 0, 0)),
            pl.BlockSpec((AS, NNEI2, 5), lambda i: (i, 0, 0)),
        ],
        out_shape=[
            jax.ShapeDtypeStruct((NLOCP, G1), jnp.float32),
            jax.ShapeDtypeStruct((NLOCP, G2D), jnp.float32),
            jax.ShapeDtypeStruct((NLOCP, NNEI2, G2D), jnp.float32),
            jax.ShapeDtypeStruct((NLOCP, NNEI2, 5), jnp.float32),
        ],
    )(rows16, own16, w0, b0, w1, b1, w2, b2, g1w3, g2iw, wg20)


def _layer_body(last, g1_ref, p_ref, pj_ref, g2_ref, env5_ref,
                wattn_ref, wg1a_ref, wg1bc_ref, bg1_ref, wg2n_ref,
                *out_refs):
    g1 = g1_ref[...]
    p = p_ref[...]
    pj = pj_ref[...]
    g2 = g2_ref[...]
    env5 = env5_ref[...]
    sw2 = env5[:, :, 0:1]
    g2a = g2 + jnp.tanh(p[:, None, :] + pj) * sw2
    q = jnp.dot(g2a.reshape(AL * NNEI2, G2D), wattn_ref[...],
                preferred_element_type=jnp.float32).reshape(AL, NNEI2, G2D)
    scores = lax.dot_general(q, g2a, (((2,), (2,)), ((0,), (0,))),
                             preferred_element_type=jnp.float32) * (
                                 1.0 / (G2D ** 0.5))
    mx = jnp.max(scores, axis=-1, keepdims=True)
    e = jnp.exp(scores - mx)
    att = e / jnp.sum(e, axis=-1, keepdims=True)
    g2b = g2a + lax.dot_general(att, g2a, (((2,), (1,)), ((0,), (0,))),
                                preferred_element_type=jnp.float32)
    m5 = lax.dot_general(env5, g2b, (((1,), (1,)), ((0,), (0,))),
                         preferred_element_type=jnp.float32) * (1.0 / NNEI2)
    acc = (jnp.dot(g1, wg1a_ref[...], preferred_element_type=jnp.float32)
           + bg1_ref[...])
    for j in range(5):
        acc = acc + jnp.dot(m5[:, j, :], wg1bc_ref[j],
                            preferred_element_type=jnp.float32)
    g1n = g1 + jnp.tanh(acc)
    out_refs[0][...] = g1n
    if not last:
        out_refs[1][...] = jnp.dot(g1n, wg2n_ref[...],
                                   preferred_element_type=jnp.float32)
        out_refs[2][...] = g2b


def _layer_call(last, g1, p, pj, g2, env5, wattn, wg1a, wg1bc, bg1r, wg2n):
    full = lambda a: pl.BlockSpec(a.shape, lambda i: (0,) * a.ndim)
    nblk = NLOCP // AL
    out_specs = [pl.BlockSpec((AL, G1), lambda i: (i, 0))]
    out_shape = [jax.ShapeDtypeStruct((NLOCP, G1), jnp.float32)]
    if not last:
        out_specs += [
            pl.BlockSpec((AL, G2D), lambda i: (i, 0)),
            pl.BlockSpec((AL, NNEI2, G2D), lambda i: (i, 0, 0)),
        ]
        out_shape += [
            jax.ShapeDtypeStruct((NLOCP, G2D), jnp.float32),
            jax.ShapeDtypeStruct((NLOCP, NNEI2, G2D), jnp.float32),
        ]
    return pl.pallas_call(
        functools.partial(_layer_body, last),
        grid=(nblk,),
        in_specs=[
            pl.BlockSpec((AL, G1), lambda i: (i, 0)),
            pl.BlockSpec((AL, G2D), lambda i: (i, 0)),
            pl.BlockSpec((AL, NNEI2, G2D), lambda i: (i, 0, 0)),
            pl.BlockSpec((AL, NNEI2, G2D), lambda i: (i, 0, 0)),
            pl.BlockSpec((AL, NNEI2, 5), lambda i: (i, 0, 0)),
            full(wattn), full(wg1a), full(wg1bc), full(bg1r), full(wg2n),
        ],
        out_specs=out_specs,
        out_shape=out_shape,
    )(g1, p, pj, g2, env5, wattn, wg1a, wg1bc, bg1r, wg2n)


def kernel(extended_coord, extended_atype, nlist, mapping, type_table, ri_w0,
           ri_b0, ri_w1, ri_b1, ri_w2, ri_b2, g1w, g2i_w, wg2, wattn, wg1,
           bg1):
    coord = extended_coord[0]
    atype2d = extended_atype[0].astype(jnp.int32).reshape(NALL, 1)
    nl = nlist[0].astype(jnp.int32)
    nlp = jnp.pad(nl, ((0, NLOCP - NLOC), (0, 0)))
    idx1 = nlp.reshape(-1)
    idx2 = nlp[:, :NNEI2].reshape(-1)
    mp = mapping[0].astype(jnp.int32)

    table16 = _build_table16(coord, atype2d, type_table)
    rows16, cidx = _sc_gather_prep()(table16, idx1, mp, idx2)

    own16 = jnp.pad(table16[:NLOC], ((0, NLOCP - NLOC), (0, 0)))
    g1w3 = g1w.reshape(N2, AXIS, G1).transpose(1, 0, 2)
    b0r, b1r, b2r = (b.reshape(1, -1) for b in (ri_b0, ri_b1, ri_b2))

    g1, p, g2, env5 = _stage1_call(
        rows16.reshape(NLOCP, NNEI, 16), own16,
        ri_w0, b0r, ri_w1, b1r, ri_w2, b2r, g1w3, g2i_w, wg2[0])

    pgather = _sc_gather_p()
    for ll in range(NL):
        pj = pgather(p, cidx).reshape(NLOCP, NNEI2, G2D)
        last = ll == NL - 1
        wg1a = wg1[ll][:G1]
        wg1bc = wg1[ll][G1:].reshape(5, G2D, G1)
        wg2n = wg2[ll + 1] if not last else wg2[0]
        outs = _layer_call(last, g1, p, pj, g2, env5, wattn[ll], wg1a,
                           wg1bc, bg1[ll].reshape(1, G1), wg2n)
        if last:
            (g1,) = outs
        else:
            g1, p, g2 = outs

    out = jnp.concatenate([g1[:NLOC], table16[:NLOC, 3:3 + TEBD]], axis=-1)
    return out[None]


# speedup vs baseline: 3.6264x; 1.0608x over previous
import functools

import jax
import jax.numpy as jnp
from jax import lax
from jax.experimental import pallas as pl
from jax.experimental.pallas import tpu as pltpu
from jax.experimental.pallas import tpu_sc as plsc

NLOC, NALL = 10000, 12000
NNEI, NNEI2 = 120, 40
NTYPES, TEBD = 8, 8
N0, N1, N2 = 25, 50, 100
AXIS = 12
G1, G2D, NL = 128, 32, 6
RC1, RS1 = 9.0, 8.0
RC2, RS2 = 4.0, 3.5

AS = 64
AL = 128
NLOCP = 10240
NC, NS = 2, 16
NW = NC * NS
E1 = NLOCP * NNEI
CH1, K1 = 1920, 20
E2 = NLOCP * NNEI2
CH2, K2 = 1600, 8


def _swfn(r, rs, rc):
    u = jnp.clip((r - rs) / (rc - rs), 0.0, 1.0)
    return u * u * u * (-6.0 * u * u + 15.0 * u - 10.0) + 1.0


def _table_body(coord_ref, atype_ref, tt_ref, out_ref):
    at = atype_ref[...]
    oh = (at == lax.broadcasted_iota(jnp.int32, (NALL, NTYPES), 1)).astype(jnp.float32)
    tebd = jnp.dot(oh, tt_ref[...], preferred_element_type=jnp.float32)
    out_ref[...] = jnp.concatenate(
        [coord_ref[...], tebd, jnp.zeros((NALL, 5), jnp.float32)], axis=1)


def _build_table16(coord, atype2d, type_table):
    return pl.pallas_call(
        _table_body,
        out_shape=jax.ShapeDtypeStruct((NALL, 16), jnp.float32),
    )(coord, atype2d, type_table)


def _sc_mesh():
    return plsc.VectorSubcoreMesh(core_axis_name="c", subcore_axis_name="s",
                                  num_cores=NC, num_subcores=NS)


def _pipelined_gather(tab, idx_v, out, base0, CH, K, bufs, gsems, ssems):
    gh = {0: pltpu.async_copy(tab.at[idx_v.at[pl.ds(0, CH)]], bufs[0],
                              gsems[0])}
    if K > 1:
        gh[1] = pltpu.async_copy(tab.at[idx_v.at[pl.ds(CH, CH)]], bufs[1],
                                 gsems[1])
    sh = {}
    for k in range(K):
        b = k % 2
        gh[k].wait()
        sh[k] = pltpu.async_copy(bufs[b], out.at[pl.ds(base0 + k * CH, CH)],
                                 ssems[b])
        if k + 2 < K:
            sh[k].wait()
            gh[k + 2] = pltpu.async_copy(
                tab.at[idx_v.at[pl.ds((k + 2) * CH, CH)]], bufs[b], gsems[b])
    for k in range(max(0, K - 2), K):
        sh[k].wait()


def _sc_gather_prep():
    def body(tab_hbm, idx1_hbm, map_hbm, idx2_hbm, rows_hbm, cidx_hbm,
             idx_v, rb0, rb1, idx2_v, cb0, cb1, gsem0, gsem1, ssem0, ssem1):
        wid = lax.axis_index("s") * NC + lax.axis_index("c")
        base1 = wid * (CH1 * K1)
        pltpu.sync_copy(idx1_hbm.at[pl.ds(base1, CH1 * K1)], idx_v)
        _pipelined_gather(tab_hbm, idx_v, rows_hbm, base1, CH1, K1,
                          (rb0, rb1), (gsem0, gsem1), (ssem0, ssem1))
        base2 = wid * (CH2 * K2)
        pltpu.sync_copy(idx2_hbm.at[pl.ds(base2, CH2 * K2)], idx2_v)
        _pipelined_gather(map_hbm, idx2_v, cidx_hbm, base2, CH2, K2,
                          (cb0, cb1), (gsem0, gsem1), (ssem0, ssem1))

    return pl.kernel(
        body,
        out_type=[jax.ShapeDtypeStruct((E1, 16), jnp.float32),
                  jax.ShapeDtypeStruct((E2,), jnp.int32)],
        mesh=_sc_mesh(),
        compiler_params=pltpu.CompilerParams(use_tc_tiling_on_sc=False),
        scratch_types=[
            pltpu.VMEM((CH1 * K1,), jnp.int32),
            pltpu.VMEM((CH1, 16), jnp.float32),
            pltpu.VMEM((CH1, 16), jnp.float32),
            pltpu.VMEM((CH2 * K2,), jnp.int32),
            pltpu.VMEM((CH2,), jnp.int32),
            pltpu.VMEM((CH2,), jnp.int32),
            pltpu.SemaphoreType.DMA,
            pltpu.SemaphoreType.DMA,
            pltpu.SemaphoreType.DMA,
            pltpu.SemaphoreType.DMA,
        ],
    )


def _sc_gather_p():
    def body(tab_hbm, idx_hbm, out_hbm, idx_v, rb0, rb1,
             gsem0, gsem1, ssem0, ssem1):
        wid = lax.axis_index("s") * NC + lax.axis_index("c")
        base0 = wid * (CH2 * K2)
        pltpu.sync_copy(idx_hbm.at[pl.ds(base0, CH2 * K2)], idx_v)
        _pipelined_gather(tab_hbm, idx_v, out_hbm, base0, CH2, K2,
                          (rb0, rb1), (gsem0, gsem1), (ssem0, ssem1))

    return pl.kernel(
        body,
        out_type=jax.ShapeDtypeStruct((E2, G2D), jnp.float32),
        mesh=_sc_mesh(),
        compiler_params=pltpu.CompilerParams(use_tc_tiling_on_sc=False),
        scratch_types=[
            pltpu.VMEM((CH2 * K2,), jnp.int32),
            pltpu.VMEM((CH2, G2D), jnp.float32),
            pltpu.VMEM((CH2, G2D), jnp.float32),
            pltpu.SemaphoreType.DMA,
            pltpu.SemaphoreType.DMA,
            pltpu.SemaphoreType.DMA,
            pltpu.SemaphoreType.DMA,
        ],
    )


def _stage1_body(rows_ref, own_ref, w0_ref, b0_ref, w1_ref, b1_ref, w2_ref,
                 b2_ref, g1w3_ref, g2iw_ref, wg20_ref,
                 g1_ref, p_ref, g2_ref, env5t_ref, sw2p_ref):
    rows = rows_ref[...]
    own = own_ref[...]
    cjT = jnp.transpose(rows[:, :, 0:3], (0, 2, 1))
    ci = own[:, 0:3]
    rijT = cjT - ci[:, :, None]
    r = jnp.sqrt(jnp.sum(rijT * rijT, axis=1, keepdims=True) + 1e-6)
    sw = _swfn(r, RS1, RC1)
    sT = sw / r
    envTw = jnp.concatenate([sT, sT * rijT / r], axis=1) * sw
    s_pair = jnp.transpose(sT, (0, 2, 1))
    tebd_i = own[:, 3:3 + TEBD]
    emb_in = jnp.concatenate(
        [s_pair, jnp.broadcast_to(tebd_i[:, None, :], (AS, NNEI, TEBD)),
         rows[:, :, 3:3 + TEBD]], axis=-1)
    h = jnp.tanh(jnp.dot(emb_in.reshape(AS * NNEI, 1 + 2 * TEBD), w0_ref[...],
                         preferred_element_type=jnp.float32) + b0_ref[...])
    h = jnp.tanh(jnp.dot(h, w1_ref[...], preferred_element_type=jnp.float32)
                 + b1_ref[...])
    h = jnp.tanh(jnp.dot(h, w2_ref[...], preferred_element_type=jnp.float32)
                 + b2_ref[...])
    h3 = h.reshape(AS, NNEI, N2)
    grr = lax.dot_general(envTw, h3, (((2,), (1,)), ((0,), (0,))),
                          preferred_element_type=jnp.float32) * (1.0 / NNEI)
    grr_ax = grr[:, :, :AXIS]
    desc2 = lax.dot_general(grr_ax, grr, (((1,), (1,)), ((0,), (0,))),
                            preferred_element_type=jnp.float32)
    g1 = jnp.zeros((AS, G1), jnp.float32)
    for x in range(AXIS):
        g1 = g1 + jnp.dot(desc2[:, x, :], g1w3_ref[x],
                          preferred_element_type=jnp.float32)
    g1_ref[...] = g1
    p_ref[...] = jnp.dot(g1, wg20_ref[...], preferred_element_type=jnp.float32)
    rij2T = rijT[:, :, :NNEI2]
    r2 = r[:, :, :NNEI2]
    sw2 = _swfn(r2, RS2, RC2)
    s2 = sw2 / r2
    env2T = jnp.concatenate([s2, s2 * rij2T / r2], axis=1)
    env5t_ref[...] = jnp.concatenate([sw2, env2T], axis=1)
    sw2p_ref[...] = jnp.transpose(sw2, (0, 2, 1))
    env2p = jnp.transpose(env2T, (0, 2, 1))
    g2_ref[...] = jnp.tanh(
        jnp.dot(env2p.reshape(AS * NNEI2, 4), g2iw_ref[...],
                preferred_element_type=jnp.float32)).reshape(AS, NNEI2, G2D)


def _stage1_call(rows16, own16, w0, b0, w1, b1, w2, b2, g1w3, g2iw, wg20):
    full = lambda a: pl.BlockSpec(a.shape, lambda i: (0,) * a.ndim)
    nblk = NLOCP // AS
    return pl.pallas_call(
        _stage1_body,
        grid=(nblk,),
        in_specs=[
            pl.BlockSpec((AS, NNEI, 16), lambda i: (i, 0, 0)),
            pl.BlockSpec((AS, 16), lambda i: (i, 0)),
            full(w0), full(b0), full(w1), full(b1), full(w2), full(b2),
            full(g1w3), full(g2iw), full(wg20),
        ],
        out_specs=[
            pl.BlockSpec((AS, G1), lambda i: (i, 0)),
            pl.BlockSpec((AS, G2D), lambda i: (i, 0)),
            pl.BlockSpec((AS, NNEI2, G2D), lambda i: (i, 0, 0)),
            pl.BlockSpec((AS, 5, NNEI2), lambda i: (i, 0, 0)),
            pl.BlockSpec((AS, NNEI2, 1), lambda i: (i, 0, 0)),
        ],
        out_shape=[
            jax.ShapeDtypeStruct((NLOCP, G1), jnp.float32),
            jax.ShapeDtypeStruct((NLOCP, G2D), jnp.float32),
            jax.ShapeDtypeStruct((NLOCP, NNEI2, G2D), jnp.float32),
            jax.ShapeDtypeStruct((NLOCP, 5, NNEI2), jnp.float32),
            jax.ShapeDtypeStruct((NLOCP, NNEI2, 1), jnp.float32),
        ],
    )(rows16, own16, w0, b0, w1, b1, w2, b2, g1w3, g2iw, wg20)


def _layer_body(last, g1_ref, p_ref, pj_ref, g2_ref, env5t_ref, sw2p_ref,
                wattn_ref, wg1a_ref, wg1bc_ref, bg1_ref, wg2n_ref,
                *out_refs):
    g1 = g1_ref[...]
    p = p_ref[...]
    pj = pj_ref[...]
    g2 = g2_ref[...]
    env5t = env5t_ref[...]
    sw2 = sw2p_ref[...]
    g2a = g2 + jnp.tanh(p[:, None, :] + pj) * sw2
    q = jnp.dot(g2a.reshape(AL * NNEI2, G2D), wattn_ref[...],
                preferred_element_type=jnp.float32).reshape(AL, NNEI2, G2D)
    scores = lax.dot_general(q, g2a, (((2,), (2,)), ((0,), (0,))),
                             preferred_element_type=jnp.float32) * (
                                 1.0 / (G2D ** 0.5))
    mx = jnp.max(scores, axis=-1, keepdims=True)
    e = jnp.exp(scores - mx)
    att = e / jnp.sum(e, axis=-1, keepdims=True)
    g2b = g2a + lax.dot_general(att, g2a, (((2,), (1,)), ((0,), (0,))),
                                preferred_element_type=jnp.float32)
    m5 = lax.dot_general(env5t, g2b, (((2,), (1,)), ((0,), (0,))),
                         preferred_element_type=jnp.float32) * (1.0 / NNEI2)
    acc = (jnp.dot(g1, wg1a_ref[...], preferred_element_type=jnp.float32)
           + bg1_ref[...])
    for j in range(5):
        acc = acc + jnp.dot(m5[:, j, :], wg1bc_ref[j],
                            preferred_element_type=jnp.float32)
    g1n = g1 + jnp.tanh(acc)
    out_refs[0][...] = g1n
    if not last:
        out_refs[1][...] = jnp.dot(g1n, wg2n_ref[...],
                                   preferred_element_type=jnp.float32)
        out_refs[2][...] = g2b


def _layer_call(last, g1, p, pj, g2, env5t, sw2p, wattn, wg1a, wg1bc, bg1r,
                wg2n):
    full = lambda a: pl.BlockSpec(a.shape, lambda i: (0,) * a.ndim)
    nblk = NLOCP // AL
    out_specs = [pl.BlockSpec((AL, G1), lambda i: (i, 0))]
    out_shape = [jax.ShapeDtypeStruct((NLOCP, G1), jnp.float32)]
    if not last:
        out_specs += [
            pl.BlockSpec((AL, G2D), lambda i: (i, 0)),
            pl.BlockSpec((AL, NNEI2, G2D), lambda i: (i, 0, 0)),
        ]
        out_shape += [
            jax.ShapeDtypeStruct((NLOCP, G2D), jnp.float32),
            jax.ShapeDtypeStruct((NLOCP, NNEI2, G2D), jnp.float32),
        ]
    return pl.pallas_call(
        functools.partial(_layer_body, last),
        grid=(nblk,),
        in_specs=[
            pl.BlockSpec((AL, G1), lambda i: (i, 0)),
            pl.BlockSpec((AL, G2D), lambda i: (i, 0)),
            pl.BlockSpec((AL, NNEI2, G2D), lambda i: (i, 0, 0)),
            pl.BlockSpec((AL, NNEI2, G2D), lambda i: (i, 0, 0)),
            pl.BlockSpec((AL, 5, NNEI2), lambda i: (i, 0, 0)),
            pl.BlockSpec((AL, NNEI2, 1), lambda i: (i, 0, 0)),
            full(wattn), full(wg1a), full(wg1bc), full(bg1r), full(wg2n),
        ],
        out_specs=out_specs,
        out_shape=out_shape,
    )(g1, p, pj, g2, env5t, sw2p, wattn, wg1a, wg1bc, bg1r, wg2n)


def kernel(extended_coord, extended_atype, nlist, mapping, type_table, ri_w0,
           ri_b0, ri_w1, ri_b1, ri_w2, ri_b2, g1w, g2i_w, wg2, wattn, wg1,
           bg1):
    coord = extended_coord[0]
    atype2d = extended_atype[0].astype(jnp.int32).reshape(NALL, 1)
    nl = nlist[0].astype(jnp.int32)
    nlp = jnp.pad(nl, ((0, NLOCP - NLOC), (0, 0)))
    idx1 = nlp.reshape(-1)
    idx2 = nlp[:, :NNEI2].reshape(-1)
    mp = mapping[0].astype(jnp.int32)

    table16 = _build_table16(coord, atype2d, type_table)
    rows16, cidx = _sc_gather_prep()(table16, idx1, mp, idx2)

    own16 = jnp.pad(table16[:NLOC], ((0, NLOCP - NLOC), (0, 0)))
    g1w3 = g1w.reshape(N2, AXIS, G1).transpose(1, 0, 2)
    b0r, b1r, b2r = (b.reshape(1, -1) for b in (ri_b0, ri_b1, ri_b2))

    g1, p, g2, env5t, sw2p = _stage1_call(
        rows16.reshape(NLOCP, NNEI, 16), own16,
        ri_w0, b0r, ri_w1, b1r, ri_w2, b2r, g1w3, g2i_w, wg2[0])

    pgather = _sc_gather_p()
    for ll in range(NL):
        pj = pgather(p, cidx).reshape(NLOCP, NNEI2, G2D)
        last = ll == NL - 1
        wg1a = wg1[ll][:G1]
        wg1bc = wg1[ll][G1:].reshape(5, G2D, G1)
        wg2n = wg2[ll + 1] if not last else wg2[0]
        outs = _layer_call(last, g1, p, pj, g2, env5t, sw2p, wattn[ll], wg1a,
                           wg1bc, bg1[ll].reshape(1, G1), wg2n)
        if last:
            (g1,) = outs
        else:
            g1, p, g2 = outs

    out = jnp.concatenate([g1[:NLOC], table16[:NLOC, 3:3 + TEBD]], axis=-1)
    return out[None]


# speedup vs baseline: 3.8172x; 1.0526x over previous
import functools

import jax
import jax.numpy as jnp
from jax import lax
from jax.experimental import pallas as pl
from jax.experimental.pallas import tpu as pltpu
from jax.experimental.pallas import tpu_sc as plsc

NLOC, NALL = 10000, 12000
NNEI, NNEI2 = 120, 40
NTYPES, TEBD = 8, 8
N0, N1, N2 = 25, 50, 100
AXIS = 12
G1, G2D, NL = 128, 32, 6
RC1, RS1 = 9.0, 8.0
RC2, RS2 = 4.0, 3.5

AS = 64
AL = 256
NLOCP = 10240
NC, NS = 2, 16
NW = NC * NS
E1 = NLOCP * NNEI
CH1, K1 = 1920, 20
E2 = NLOCP * NNEI2
CH2, K2 = 1600, 8


def _swfn(r, rs, rc):
    u = jnp.clip((r - rs) / (rc - rs), 0.0, 1.0)
    return u * u * u * (-6.0 * u * u + 15.0 * u - 10.0) + 1.0


def _table_body(coord_ref, atype_ref, tt_ref, out_ref):
    at = atype_ref[...]
    oh = (at == lax.broadcasted_iota(jnp.int32, (NALL, NTYPES), 1)).astype(jnp.float32)
    tebd = jnp.dot(oh, tt_ref[...], preferred_element_type=jnp.float32)
    out_ref[...] = jnp.concatenate(
        [coord_ref[...], tebd, jnp.zeros((NALL, 5), jnp.float32)], axis=1)


def _build_table16(coord, atype2d, type_table):
    return pl.pallas_call(
        _table_body,
        out_shape=jax.ShapeDtypeStruct((NALL, 16), jnp.float32),
    )(coord, atype2d, type_table)


def _sc_mesh():
    return plsc.VectorSubcoreMesh(core_axis_name="c", subcore_axis_name="s",
                                  num_cores=NC, num_subcores=NS)


def _pipelined_gather(tab, idx_v, out, base0, CH, K, bufs, gsems, ssems):
    gh = {0: pltpu.async_copy(tab.at[idx_v.at[pl.ds(0, CH)]], bufs[0],
                              gsems[0])}
    if K > 1:
        gh[1] = pltpu.async_copy(tab.at[idx_v.at[pl.ds(CH, CH)]], bufs[1],
                                 gsems[1])
    sh = {}
    for k in range(K):
        b = k % 2
        gh[k].wait()
        sh[k] = pltpu.async_copy(bufs[b], out.at[pl.ds(base0 + k * CH, CH)],
                                 ssems[b])
        if k + 2 < K:
            sh[k].wait()
            gh[k + 2] = pltpu.async_copy(
                tab.at[idx_v.at[pl.ds((k + 2) * CH, CH)]], bufs[b], gsems[b])
    for k in range(max(0, K - 2), K):
        sh[k].wait()


def _sc_gather_prep():
    def body(tab_hbm, idx1_hbm, rows_hbm,
             idx_v, rb0, rb1, gsem0, gsem1, ssem0, ssem1):
        wid = lax.axis_index("s") * NC + lax.axis_index("c")
        base1 = wid * (CH1 * K1)
        pltpu.sync_copy(idx1_hbm.at[pl.ds(base1, CH1 * K1)], idx_v)
        _pipelined_gather(tab_hbm, idx_v, rows_hbm, base1, CH1, K1,
                          (rb0, rb1), (gsem0, gsem1), (ssem0, ssem1))

    return pl.kernel(
        body,
        out_type=jax.ShapeDtypeStruct((E1, 16), jnp.float32),
        mesh=_sc_mesh(),
        compiler_params=pltpu.CompilerParams(use_tc_tiling_on_sc=False),
        scratch_types=[
            pltpu.VMEM((CH1 * K1,), jnp.int32),
            pltpu.VMEM((CH1, 16), jnp.float32),
            pltpu.VMEM((CH1, 16), jnp.float32),
            pltpu.SemaphoreType.DMA,
            pltpu.SemaphoreType.DMA,
            pltpu.SemaphoreType.DMA,
            pltpu.SemaphoreType.DMA,
        ],
    )


def _sc_gather_cidx():
    def body(map_hbm, idx2_hbm, cidx_hbm,
             idx2_v, cb0, cb1, gsem0, gsem1, ssem0, ssem1):
        wid = lax.axis_index("s") * NC + lax.axis_index("c")
        base2 = wid * (CH2 * K2)
        pltpu.sync_copy(idx2_hbm.at[pl.ds(base2, CH2 * K2)], idx2_v)
        _pipelined_gather(map_hbm, idx2_v, cidx_hbm, base2, CH2, K2,
                          (cb0, cb1), (gsem0, gsem1), (ssem0, ssem1))

    return pl.kernel(
        body,
        out_type=jax.ShapeDtypeStruct((E2,), jnp.int32),
        mesh=_sc_mesh(),
        compiler_params=pltpu.CompilerParams(use_tc_tiling_on_sc=False),
        scratch_types=[
            pltpu.VMEM((CH2 * K2,), jnp.int32),
            pltpu.VMEM((CH2,), jnp.int32),
            pltpu.VMEM((CH2,), jnp.int32),
            pltpu.SemaphoreType.DMA,
            pltpu.SemaphoreType.DMA,
            pltpu.SemaphoreType.DMA,
            pltpu.SemaphoreType.DMA,
        ],
    )


def _sc_gather_p():
    def body(tab_hbm, idx_hbm, out_hbm, idx_v, rb0, rb1,
             gsem0, gsem1, ssem0, ssem1):
        wid = lax.axis_index("s") * NC + lax.axis_index("c")
        base0 = wid * (CH2 * K2)
        pltpu.sync_copy(idx_hbm.at[pl.ds(base0, CH2 * K2)], idx_v)
        _pipelined_gather(tab_hbm, idx_v, out_hbm, base0, CH2, K2,
                          (rb0, rb1), (gsem0, gsem1), (ssem0, ssem1))

    return pl.kernel(
        body,
        out_type=jax.ShapeDtypeStruct((E2, G2D), jnp.float32),
        mesh=_sc_mesh(),
        compiler_params=pltpu.CompilerParams(use_tc_tiling_on_sc=False),
        scratch_types=[
            pltpu.VMEM((CH2 * K2,), jnp.int32),
            pltpu.VMEM((CH2, G2D), jnp.float32),
            pltpu.VMEM((CH2, G2D), jnp.float32),
            pltpu.SemaphoreType.DMA,
            pltpu.SemaphoreType.DMA,
            pltpu.SemaphoreType.DMA,
            pltpu.SemaphoreType.DMA,
        ],
    )


def _stage1_body(rows_ref, own_ref, w0_ref, b0_ref, w1_ref, b1_ref, w2_ref,
                 b2_ref, g1w3_ref, g2iw_ref, wg20_ref,
                 g1_ref, p_ref, g2_ref, env5t_ref, sw2p_ref):
    rows = rows_ref[...]
    own = own_ref[...]
    cjT = jnp.transpose(rows[:, :, 0:3], (0, 2, 1))
    ci = own[:, 0:3]
    rijT = cjT - ci[:, :, None]
    r = jnp.sqrt(jnp.sum(rijT * rijT, axis=1, keepdims=True) + 1e-6)
    sw = _swfn(r, RS1, RC1)
    sT = sw / r
    envTw = jnp.concatenate([sT, sT * rijT / r], axis=1) * sw
    s_pair = jnp.transpose(sT, (0, 2, 1))
    tebd_i = own[:, 3:3 + TEBD]
    emb_in = jnp.concatenate(
        [s_pair, jnp.broadcast_to(tebd_i[:, None, :], (AS, NNEI, TEBD)),
         rows[:, :, 3:3 + TEBD]], axis=-1)
    h = jnp.tanh(jnp.dot(emb_in.reshape(AS * NNEI, 1 + 2 * TEBD), w0_ref[...],
                         preferred_element_type=jnp.float32) + b0_ref[...])
    h = jnp.tanh(jnp.dot(h, w1_ref[...], preferred_element_type=jnp.float32)
                 + b1_ref[...])
    h = jnp.tanh(jnp.dot(h, w2_ref[...], preferred_element_type=jnp.float32)
                 + b2_ref[...])
    h3 = h.reshape(AS, NNEI, N2)
    grr = lax.dot_general(envTw, h3, (((2,), (1,)), ((0,), (0,))),
                          preferred_element_type=jnp.float32) * (1.0 / NNEI)
    grr_ax = grr[:, :, :AXIS]
    desc2 = lax.dot_general(grr_ax, grr, (((1,), (1,)), ((0,), (0,))),
                            preferred_element_type=jnp.float32)
    accs = [jnp.zeros((AS, G1), jnp.float32) for _ in range(4)]
    for x in range(AXIS):
        accs[x % 4] = accs[x % 4] + jnp.dot(
            desc2[:, x, :], g1w3_ref[x], preferred_element_type=jnp.float32)
    g1 = (accs[0] + accs[1]) + (accs[2] + accs[3])
    g1_ref[...] = g1
    p_ref[...] = jnp.dot(g1, wg20_ref[...], preferred_element_type=jnp.float32)
    rij2T = rijT[:, :, :NNEI2]
    r2 = r[:, :, :NNEI2]
    sw2 = _swfn(r2, RS2, RC2)
    s2 = sw2 / r2
    env2T = jnp.concatenate([s2, s2 * rij2T / r2], axis=1)
    env5t_ref[...] = jnp.concatenate([sw2, env2T], axis=1)
    sw2p_ref[...] = jnp.transpose(sw2, (0, 2, 1))
    env2p = jnp.transpose(env2T, (0, 2, 1))
    g2_ref[...] = jnp.tanh(
        jnp.dot(env2p.reshape(AS * NNEI2, 4), g2iw_ref[...],
                preferred_element_type=jnp.float32)).reshape(AS, NNEI2, G2D)


def _stage1_call(rows16, own16, w0, b0, w1, b1, w2, b2, g1w3, g2iw, wg20):
    full = lambda a: pl.BlockSpec(a.shape, lambda i: (0,) * a.ndim)
    nblk = NLOCP // AS
    return pl.pallas_call(
        _stage1_body,
        grid=(nblk,),
        in_specs=[
            pl.BlockSpec((AS, NNEI, 16), lambda i: (i, 0, 0)),
            pl.BlockSpec((AS, 16), lambda i: (i, 0)),
            full(w0), full(b0), full(w1), full(b1), full(w2), full(b2),
            full(g1w3), full(g2iw), full(wg20),
        ],
        out_specs=[
            pl.BlockSpec((AS, G1), lambda i: (i, 0)),
            pl.BlockSpec((AS, G2D), lambda i: (i, 0)),
            pl.BlockSpec((AS, NNEI2, G2D), lambda i: (i, 0, 0)),
            pl.BlockSpec((AS, 5, NNEI2), lambda i: (i, 0, 0)),
            pl.BlockSpec((AS, NNEI2, 1), lambda i: (i, 0, 0)),
        ],
        out_shape=[
            jax.ShapeDtypeStruct((NLOCP, G1), jnp.float32),
            jax.ShapeDtypeStruct((NLOCP, G2D), jnp.float32),
            jax.ShapeDtypeStruct((NLOCP, NNEI2, G2D), jnp.float32),
            jax.ShapeDtypeStruct((NLOCP, 5, NNEI2), jnp.float32),
            jax.ShapeDtypeStruct((NLOCP, NNEI2, 1), jnp.float32),
        ],
    )(rows16, own16, w0, b0, w1, b1, w2, b2, g1w3, g2iw, wg20)


def _layer_body(last, g1_ref, p_ref, pj_ref, g2_ref, env5t_ref, sw2p_ref,
                wattn_ref, wg1a_ref, wg1bc_ref, bg1_ref, wg2n_ref,
                *out_refs):
    g1 = g1_ref[...]
    p = p_ref[...]
    pj = pj_ref[...]
    g2 = g2_ref[...]
    env5t = env5t_ref[...]
    sw2 = sw2p_ref[...]
    g2a = g2 + jnp.tanh(p[:, None, :] + pj) * sw2
    q = jnp.dot(g2a.reshape(AL * NNEI2, G2D), wattn_ref[...],
                preferred_element_type=jnp.float32).reshape(AL, NNEI2, G2D)
    scores = lax.dot_general(q, g2a, (((2,), (2,)), ((0,), (0,))),
                             preferred_element_type=jnp.float32) * (
                                 1.0 / (G2D ** 0.5))
    mx = jnp.max(scores, axis=-1, keepdims=True)
    e = jnp.exp(scores - mx)
    att = e / jnp.sum(e, axis=-1, keepdims=True)
    g2b = g2a + lax.dot_general(att, g2a, (((2,), (1,)), ((0,), (0,))),
                                preferred_element_type=jnp.float32)
    m5 = lax.dot_general(env5t, g2b, (((2,), (1,)), ((0,), (0,))),
                         preferred_element_type=jnp.float32) * (1.0 / NNEI2)
    acc0 = (jnp.dot(g1, wg1a_ref[...], preferred_element_type=jnp.float32)
            + bg1_ref[...])
    acc1 = jnp.dot(m5[:, 0, :], wg1bc_ref[0],
                   preferred_element_type=jnp.float32)
    acc2 = jnp.dot(m5[:, 1, :], wg1bc_ref[1],
                   preferred_element_type=jnp.float32)
    for j in range(2, 5):
        a = jnp.dot(m5[:, j, :], wg1bc_ref[j],
                    preferred_element_type=jnp.float32)
        if j % 2 == 0:
            acc1 = acc1 + a
        else:
            acc2 = acc2 + a
    g1n = g1 + jnp.tanh(acc0 + (acc1 + acc2))
    out_refs[0][...] = g1n
    if not last:
        out_refs[1][...] = jnp.dot(g1n, wg2n_ref[...],
                                   preferred_element_type=jnp.float32)
        out_refs[2][...] = g2b


def _layer_call(last, g1, p, pj, g2, env5t, sw2p, wattn, wg1a, wg1bc, bg1r,
                wg2n):
    full = lambda a: pl.BlockSpec(a.shape, lambda i: (0,) * a.ndim)
    nblk = NLOCP // AL
    out_specs = [pl.BlockSpec((AL, G1), lambda i: (i, 0))]
    out_shape = [jax.ShapeDtypeStruct((NLOCP, G1), jnp.float32)]
    if not last:
        out_specs += [
            pl.BlockSpec((AL, G2D), lambda i: (i, 0)),
            pl.BlockSpec((AL, NNEI2, G2D), lambda i: (i, 0, 0)),
        ]
        out_shape += [
            jax.ShapeDtypeStruct((NLOCP, G2D), jnp.float32),
            jax.ShapeDtypeStruct((NLOCP, NNEI2, G2D), jnp.float32),
        ]
    return pl.pallas_call(
        functools.partial(_layer_body, last),
        grid=(nblk,),
        in_specs=[
            pl.BlockSpec((AL, G1), lambda i: (i, 0)),
            pl.BlockSpec((AL, G2D), lambda i: (i, 0)),
            pl.BlockSpec((AL, NNEI2, G2D), lambda i: (i, 0, 0)),
            pl.BlockSpec((AL, NNEI2, G2D), lambda i: (i, 0, 0)),
            pl.BlockSpec((AL, 5, NNEI2), lambda i: (i, 0, 0)),
            pl.BlockSpec((AL, NNEI2, 1), lambda i: (i, 0, 0)),
            full(wattn), full(wg1a), full(wg1bc), full(bg1r), full(wg2n),
        ],
        out_specs=out_specs,
        out_shape=out_shape,
    )(g1, p, pj, g2, env5t, sw2p, wattn, wg1a, wg1bc, bg1r, wg2n)


def kernel(extended_coord, extended_atype, nlist, mapping, type_table, ri_w0,
           ri_b0, ri_w1, ri_b1, ri_w2, ri_b2, g1w, g2i_w, wg2, wattn, wg1,
           bg1):
    coord = extended_coord[0]
    atype2d = extended_atype[0].astype(jnp.int32).reshape(NALL, 1)
    nl = nlist[0].astype(jnp.int32)
    nlp = jnp.pad(nl, ((0, NLOCP - NLOC), (0, 0)))
    idx1 = nlp.reshape(-1)
    idx2 = nlp[:, :NNEI2].reshape(-1)
    mp = mapping[0].astype(jnp.int32)

    table16 = _build_table16(coord, atype2d, type_table)
    rows16 = _sc_gather_prep()(table16, idx1)
    cidx = _sc_gather_cidx()(mp, idx2)

    own16 = jnp.pad(table16[:NLOC], ((0, NLOCP - NLOC), (0, 0)))
    g1w3 = g1w.reshape(N2, AXIS, G1).transpose(1, 0, 2)
    b0r, b1r, b2r = (b.reshape(1, -1) for b in (ri_b0, ri_b1, ri_b2))

    g1, p, g2, env5t, sw2p = _stage1_call(
        rows16.reshape(NLOCP, NNEI, 16), own16,
        ri_w0, b0r, ri_w1, b1r, ri_w2, b2r, g1w3, g2i_w, wg2[0])

    pgather = _sc_gather_p()
    for ll in range(NL):
        pj = pgather(p, cidx).reshape(NLOCP, NNEI2, G2D)
        last = ll == NL - 1
        wg1a = wg1[ll][:G1]
        wg1bc = wg1[ll][G1:].reshape(5, G2D, G1)
        wg2n = wg2[ll + 1] if not last else wg2[0]
        outs = _layer_call(last, g1, p, pj, g2, env5t, sw2p, wattn[ll], wg1a,
                           wg1bc, bg1[ll].reshape(1, G1), wg2n)
        if last:
            (g1,) = outs
        else:
            g1, p, g2 = outs

    out = jnp.concatenate([g1[:NLOC], table16[:NLOC, 3:3 + TEBD]], axis=-1)
    return out[None]


# speedup vs baseline: 3.8459x; 1.0075x over previous
import functools

import jax
import jax.numpy as jnp
from jax import lax
from jax.experimental import pallas as pl
from jax.experimental.pallas import tpu as pltpu
from jax.experimental.pallas import tpu_sc as plsc

NLOC, NALL = 10000, 12000
NNEI, NNEI2 = 120, 40
NTYPES, TEBD = 8, 8
N0, N1, N2 = 25, 50, 100
AXIS = 12
G1, G2D, NL = 128, 32, 6
RC1, RS1 = 9.0, 8.0
RC2, RS2 = 4.0, 3.5

AS = 128
AL = 256
NLOCP = 10240
NC, NS = 2, 16
NW = NC * NS
E1 = NLOCP * NNEI
CH1, K1 = 1920, 20
E2 = NLOCP * NNEI2
CH2, K2 = 1600, 8


def _swfn(r, rs, rc):
    u = jnp.clip((r - rs) / (rc - rs), 0.0, 1.0)
    return u * u * u * (-6.0 * u * u + 15.0 * u - 10.0) + 1.0


def _table_body(coord_ref, atype_ref, tt_ref, out_ref):
    at = atype_ref[...]
    oh = (at == lax.broadcasted_iota(jnp.int32, (NALL, NTYPES), 1)).astype(jnp.float32)
    tebd = jnp.dot(oh, tt_ref[...], preferred_element_type=jnp.float32)
    out_ref[...] = jnp.concatenate(
        [coord_ref[...], tebd, jnp.zeros((NALL, 5), jnp.float32)], axis=1)


def _build_table16(coord, atype2d, type_table):
    return pl.pallas_call(
        _table_body,
        out_shape=jax.ShapeDtypeStruct((NALL, 16), jnp.float32),
    )(coord, atype2d, type_table)


def _sc_mesh():
    return plsc.VectorSubcoreMesh(core_axis_name="c", subcore_axis_name="s",
                                  num_cores=NC, num_subcores=NS)


def _pipelined_gather(tab, idx_v, out, base0, CH, K, bufs, gsems, ssems):
    gh = {0: pltpu.async_copy(tab.at[idx_v.at[pl.ds(0, CH)]], bufs[0],
                              gsems[0])}
    if K > 1:
        gh[1] = pltpu.async_copy(tab.at[idx_v.at[pl.ds(CH, CH)]], bufs[1],
                                 gsems[1])
    sh = {}
    for k in range(K):
        b = k % 2
        gh[k].wait()
        sh[k] = pltpu.async_copy(bufs[b], out.at[pl.ds(base0 + k * CH, CH)],
                                 ssems[b])
        if k + 2 < K:
            sh[k].wait()
            gh[k + 2] = pltpu.async_copy(
                tab.at[idx_v.at[pl.ds((k + 2) * CH, CH)]], bufs[b], gsems[b])
    for k in range(max(0, K - 2), K):
        sh[k].wait()


def _sc_gather_prep():
    def body(tab_hbm, idx1_hbm, rows_hbm,
             idx_v, rb0, rb1, gsem0, gsem1, ssem0, ssem1):
        wid = lax.axis_index("s") * NC + lax.axis_index("c")
        base1 = wid * (CH1 * K1)
        pltpu.sync_copy(idx1_hbm.at[pl.ds(base1, CH1 * K1)], idx_v)
        _pipelined_gather(tab_hbm, idx_v, rows_hbm, base1, CH1, K1,
                          (rb0, rb1), (gsem0, gsem1), (ssem0, ssem1))

    return pl.kernel(
        body,
        out_type=jax.ShapeDtypeStruct((E1, 16), jnp.float32),
        mesh=_sc_mesh(),
        compiler_params=pltpu.CompilerParams(use_tc_tiling_on_sc=False),
        scratch_types=[
            pltpu.VMEM((CH1 * K1,), jnp.int32),
            pltpu.VMEM((CH1, 16), jnp.float32),
            pltpu.VMEM((CH1, 16), jnp.float32),
            pltpu.SemaphoreType.DMA,
            pltpu.SemaphoreType.DMA,
            pltpu.SemaphoreType.DMA,
            pltpu.SemaphoreType.DMA,
        ],
    )


def _sc_gather_cidx():
    def body(map_hbm, idx2_hbm, cidx_hbm,
             idx2_v, cb0, cb1, gsem0, gsem1, ssem0, ssem1):
        wid = lax.axis_index("s") * NC + lax.axis_index("c")
        base2 = wid * (CH2 * K2)
        pltpu.sync_copy(idx2_hbm.at[pl.ds(base2, CH2 * K2)], idx2_v)
        _pipelined_gather(map_hbm, idx2_v, cidx_hbm, base2, CH2, K2,
                          (cb0, cb1), (gsem0, gsem1), (ssem0, ssem1))

    return pl.kernel(
        body,
        out_type=jax.ShapeDtypeStruct((E2,), jnp.int32),
        mesh=_sc_mesh(),
        compiler_params=pltpu.CompilerParams(use_tc_tiling_on_sc=False),
        scratch_types=[
            pltpu.VMEM((CH2 * K2,), jnp.int32),
            pltpu.VMEM((CH2,), jnp.int32),
            pltpu.VMEM((CH2,), jnp.int32),
            pltpu.SemaphoreType.DMA,
            pltpu.SemaphoreType.DMA,
            pltpu.SemaphoreType.DMA,
            pltpu.SemaphoreType.DMA,
        ],
    )


def _sc_gather_p():
    def body(tab_hbm, idx_hbm, out_hbm, idx_v, rb0, rb1,
             gsem0, gsem1, ssem0, ssem1):
        wid = lax.axis_index("s") * NC + lax.axis_index("c")
        base0 = wid * (CH2 * K2)
        pltpu.sync_copy(idx_hbm.at[pl.ds(base0, CH2 * K2)], idx_v)
        _pipelined_gather(tab_hbm, idx_v, out_hbm, base0, CH2, K2,
                          (rb0, rb1), (gsem0, gsem1), (ssem0, ssem1))

    return pl.kernel(
        body,
        out_type=jax.ShapeDtypeStruct((E2, G2D), jnp.float32),
        mesh=_sc_mesh(),
        compiler_params=pltpu.CompilerParams(use_tc_tiling_on_sc=False),
        scratch_types=[
            pltpu.VMEM((CH2 * K2,), jnp.int32),
            pltpu.VMEM((CH2, G2D), jnp.float32),
            pltpu.VMEM((CH2, G2D), jnp.float32),
            pltpu.SemaphoreType.DMA,
            pltpu.SemaphoreType.DMA,
            pltpu.SemaphoreType.DMA,
            pltpu.SemaphoreType.DMA,
        ],
    )


def _stage1_body(rows_ref, own_ref, w0_ref, b0_ref, w1_ref, b1_ref, w2_ref,
                 b2_ref, g1w3_ref, g2iw_ref, wg20_ref,
                 g1_ref, p_ref, g2_ref, env5t_ref, sw2p_ref):
    rows = rows_ref[...]
    own = own_ref[...]
    cjT = jnp.transpose(rows[:, :, 0:3], (0, 2, 1))
    ci = own[:, 0:3]
    rijT = cjT - ci[:, :, None]
    r = jnp.sqrt(jnp.sum(rijT * rijT, axis=1, keepdims=True) + 1e-6)
    sw = _swfn(r, RS1, RC1)
    sT = sw / r
    envTw = jnp.concatenate([sT, sT * rijT / r], axis=1) * sw
    s_pair = jnp.transpose(sT, (0, 2, 1))
    tebd_i = own[:, 3:3 + TEBD]
    emb_in = jnp.concatenate(
        [s_pair, jnp.broadcast_to(tebd_i[:, None, :], (AS, NNEI, TEBD)),
         rows[:, :, 3:3 + TEBD]], axis=-1)
    h = jnp.tanh(jnp.dot(emb_in.reshape(AS * NNEI, 1 + 2 * TEBD), w0_ref[...],
                         preferred_element_type=jnp.float32) + b0_ref[...])
    h = jnp.tanh(jnp.dot(h, w1_ref[...], preferred_element_type=jnp.float32)
                 + b1_ref[...])
    h = jnp.tanh(jnp.dot(h, w2_ref[...], preferred_element_type=jnp.float32)
                 + b2_ref[...])
    h3 = h.reshape(AS, NNEI, N2)
    grr = lax.dot_general(envTw, h3, (((2,), (1,)), ((0,), (0,))),
                          preferred_element_type=jnp.float32) * (1.0 / NNEI)
    grr_ax = grr[:, :, :AXIS]
    desc2 = lax.dot_general(grr_ax, grr, (((1,), (1,)), ((0,), (0,))),
                            preferred_element_type=jnp.float32)
    accs = [jnp.zeros((AS, G1), jnp.float32) for _ in range(4)]
    for x in range(AXIS):
        accs[x % 4] = accs[x % 4] + jnp.dot(
            desc2[:, x, :], g1w3_ref[x], preferred_element_type=jnp.float32)
    g1 = (accs[0] + accs[1]) + (accs[2] + accs[3])
    g1_ref[...] = g1
    p_ref[...] = jnp.dot(g1, wg20_ref[...], preferred_element_type=jnp.float32)
    rij2T = rijT[:, :, :NNEI2]
    r2 = r[:, :, :NNEI2]
    sw2 = _swfn(r2, RS2, RC2)
    s2 = sw2 / r2
    env2T = jnp.concatenate([s2, s2 * rij2T / r2], axis=1)
    env5t_ref[...] = jnp.concatenate([sw2, env2T], axis=1)
    sw2p_ref[...] = jnp.transpose(sw2, (0, 2, 1))
    env2p = jnp.transpose(env2T, (0, 2, 1))
    g2_ref[...] = jnp.tanh(
        jnp.dot(env2p.reshape(AS * NNEI2, 4), g2iw_ref[...],
                preferred_element_type=jnp.float32)).reshape(AS, NNEI2, G2D)


def _stage1_call(rows16, own16, w0, b0, w1, b1, w2, b2, g1w3, g2iw, wg20):
    full = lambda a: pl.BlockSpec(a.shape, lambda i: (0,) * a.ndim)
    nblk = NLOCP // AS
    return pl.pallas_call(
        _stage1_body,
        grid=(nblk,),
        compiler_params=pltpu.CompilerParams(
            vmem_limit_bytes=100 * 1024 * 1024),
        in_specs=[
            pl.BlockSpec((AS, NNEI, 16), lambda i: (i, 0, 0)),
            pl.BlockSpec((AS, 16), lambda i: (i, 0)),
            full(w0), full(b0), full(w1), full(b1), full(w2), full(b2),
            full(g1w3), full(g2iw), full(wg20),
        ],
        out_specs=[
            pl.BlockSpec((AS, G1), lambda i: (i, 0)),
            pl.BlockSpec((AS, G2D), lambda i: (i, 0)),
            pl.BlockSpec((AS, NNEI2, G2D), lambda i: (i, 0, 0)),
            pl.BlockSpec((AS, 5, NNEI2), lambda i: (i, 0, 0)),
            pl.BlockSpec((AS, NNEI2, 1), lambda i: (i, 0, 0)),
        ],
        out_shape=[
            jax.ShapeDtypeStruct((NLOCP, G1), jnp.float32),
            jax.ShapeDtypeStruct((NLOCP, G2D), jnp.float32),
            jax.ShapeDtypeStruct((NLOCP, NNEI2, G2D), jnp.float32),
            jax.ShapeDtypeStruct((NLOCP, 5, NNEI2), jnp.float32),
            jax.ShapeDtypeStruct((NLOCP, NNEI2, 1), jnp.float32),
        ],
    )(rows16, own16, w0, b0, w1, b1, w2, b2, g1w3, g2iw, wg20)


def _layer_body(last, g1_ref, p_ref, pj_ref, g2_ref, env5t_ref, sw2p_ref,
                wattn_ref, wg1a_ref, wg1bc_ref, bg1_ref, wg2n_ref,
                *out_refs):
    g1 = g1_ref[...]
    p = p_ref[...]
    pj = pj_ref[...]
    g2 = g2_ref[...]
    env5t = env5t_ref[...]
    sw2 = sw2p_ref[...]
    g2a = g2 + jnp.tanh(p[:, None, :] + pj) * sw2
    q = jnp.dot(g2a.reshape(AL * NNEI2, G2D), wattn_ref[...],
                preferred_element_type=jnp.float32).reshape(AL, NNEI2, G2D)
    scores = lax.dot_general(q, g2a, (((2,), (2,)), ((0,), (0,))),
                             preferred_element_type=jnp.float32) * (
                                 1.0 / (G2D ** 0.5))
    mx = jnp.max(scores, axis=-1, keepdims=True)
    e = jnp.exp(scores - mx)
    att = e / jnp.sum(e, axis=-1, keepdims=True)
    g2b = g2a + lax.dot_general(att, g2a, (((2,), (1,)), ((0,), (0,))),
                                preferred_element_type=jnp.float32)
    m5 = lax.dot_general(env5t, g2b, (((2,), (1,)), ((0,), (0,))),
                         preferred_element_type=jnp.float32) * (1.0 / NNEI2)
    acc0 = (jnp.dot(g1, wg1a_ref[...], preferred_element_type=jnp.float32)
            + bg1_ref[...])
    acc1 = jnp.dot(m5[:, 0, :], wg1bc_ref[0],
                   preferred_element_type=jnp.float32)
    acc2 = jnp.dot(m5[:, 1, :], wg1bc_ref[1],
                   preferred_element_type=jnp.float32)
    for j in range(2, 5):
        a = jnp.dot(m5[:, j, :], wg1bc_ref[j],
                    preferred_element_type=jnp.float32)
        if j % 2 == 0:
            acc1 = acc1 + a
        else:
            acc2 = acc2 + a
    g1n = g1 + jnp.tanh(acc0 + (acc1 + acc2))
    out_refs[0][...] = g1n
    if not last:
        out_refs[1][...] = jnp.dot(g1n, wg2n_ref[...],
                                   preferred_element_type=jnp.float32)
        out_refs[2][...] = g2b


def _layer_call(last, g1, p, pj, g2, env5t, sw2p, wattn, wg1a, wg1bc, bg1r,
                wg2n):
    full = lambda a: pl.BlockSpec(a.shape, lambda i: (0,) * a.ndim)
    nblk = NLOCP // AL
    out_specs = [pl.BlockSpec((AL, G1), lambda i: (i, 0))]
    out_shape = [jax.ShapeDtypeStruct((NLOCP, G1), jnp.float32)]
    if not last:
        out_specs += [
            pl.BlockSpec((AL, G2D), lambda i: (i, 0)),
            pl.BlockSpec((AL, NNEI2, G2D), lambda i: (i, 0, 0)),
        ]
        out_shape += [
            jax.ShapeDtypeStruct((NLOCP, G2D), jnp.float32),
            jax.ShapeDtypeStruct((NLOCP, NNEI2, G2D), jnp.float32),
        ]
    return pl.pallas_call(
        functools.partial(_layer_body, last),
        grid=(nblk,),
        in_specs=[
            pl.BlockSpec((AL, G1), lambda i: (i, 0)),
            pl.BlockSpec((AL, G2D), lambda i: (i, 0)),
            pl.BlockSpec((AL, NNEI2, G2D), lambda i: (i, 0, 0)),
            pl.BlockSpec((AL, NNEI2, G2D), lambda i: (i, 0, 0)),
            pl.BlockSpec((AL, 5, NNEI2), lambda i: (i, 0, 0)),
            pl.BlockSpec((AL, NNEI2, 1), lambda i: (i, 0, 0)),
            full(wattn), full(wg1a), full(wg1bc), full(bg1r), full(wg2n),
        ],
        out_specs=out_specs,
        out_shape=out_shape,
    )(g1, p, pj, g2, env5t, sw2p, wattn, wg1a, wg1bc, bg1r, wg2n)


def kernel(extended_coord, extended_atype, nlist, mapping, type_table, ri_w0,
           ri_b0, ri_w1, ri_b1, ri_w2, ri_b2, g1w, g2i_w, wg2, wattn, wg1,
           bg1):
    coord = extended_coord[0]
    atype2d = extended_atype[0].astype(jnp.int32).reshape(NALL, 1)
    nl = nlist[0].astype(jnp.int32)
    nlp = jnp.pad(nl, ((0, NLOCP - NLOC), (0, 0)))
    idx1 = nlp.reshape(-1)
    idx2 = nlp[:, :NNEI2].reshape(-1)
    mp = mapping[0].astype(jnp.int32)

    table16 = _build_table16(coord, atype2d, type_table)
    rows16 = _sc_gather_prep()(table16, idx1)
    cidx = _sc_gather_cidx()(mp, idx2)

    own16 = jnp.pad(table16[:NLOC], ((0, NLOCP - NLOC), (0, 0)))
    g1w3 = g1w.reshape(N2, AXIS, G1).transpose(1, 0, 2)
    b0r, b1r, b2r = (b.reshape(1, -1) for b in (ri_b0, ri_b1, ri_b2))

    g1, p, g2, env5t, sw2p = _stage1_call(
        rows16.reshape(NLOCP, NNEI, 16), own16,
        ri_w0, b0r, ri_w1, b1r, ri_w2, b2r, g1w3, g2i_w, wg2[0])

    pgather = _sc_gather_p()
    for ll in range(NL):
        pj = pgather(p, cidx).reshape(NLOCP, NNEI2, G2D)
        last = ll == NL - 1
        wg1a = wg1[ll][:G1]
        wg1bc = wg1[ll][G1:].reshape(5, G2D, G1)
        wg2n = wg2[ll + 1] if not last else wg2[0]
        outs = _layer_call(last, g1, p, pj, g2, env5t, sw2p, wattn[ll], wg1a,
                           wg1bc, bg1[ll].reshape(1, G1), wg2n)
        if last:
            (g1,) = outs
        else:
            g1, p, g2 = outs

    out = jnp.concatenate([g1[:NLOC], table16[:NLOC, 3:3 + TEBD]], axis=-1)
    return out[None]
